# baseline (speedup 1.0000x reference)
import jax
import jax.numpy as jnp
from jax import lax
from jax.experimental import pallas as pl
from jax.experimental.pallas import tpu as pltpu

FEATURE_DIM = 128
HIDDEN_DIM = 256
EMBED_DIM = 256
N_NODES = 10000
N_NBR = 160000

_R_CHUNK = 10000
_N_RSTEPS = N_NBR // _R_CHUNK
_ROW_TILE = 2000
_N_DSTEPS = N_NODES // _ROW_TILE


def _body(l1_ref, l2_ref, nf_ref,
          wn1_ref, bn1_ref, wn2_ref, bn2_ref,
          ws1_ref, bs1_ref, g1_ref, be1_ref,
          ws2_ref, bs2_ref, g2_ref, be2_ref,
          out_ref, acc1_ref, acc2_ref, a_ref, ssq_ref, nbr1_ref, nbr2_ref):
    i = pl.program_id(0)

    @pl.when(i == 0)
    def _init():
        acc1_ref[...] = jnp.zeros_like(acc1_ref)
        acc2_ref[...] = jnp.zeros_like(acc2_ref)
        ssq_ref[...] = jnp.zeros_like(ssq_ref)

    @pl.when(i < _N_RSTEPS)
    def _reduce():
        acc1_ref[...] += l1_ref[...].reshape(_R_CHUNK // 8, 8, FEATURE_DIM).sum(axis=0)
        acc2_ref[...] += l2_ref[...].reshape(_R_CHUNK // 8, 8, HIDDEN_DIM).sum(axis=0)

    @pl.when(i < _N_DSTEPS)
    def _precompute_a():
        a_ref[pl.ds(i * _ROW_TILE, _ROW_TILE), :] = (
            jnp.dot(nf_ref[...], ws1_ref[...],
                    preferred_element_type=jnp.float32) + bs1_ref[...])

    @pl.when(i == _N_RSTEPS - 1)
    def _finalize_aggs():
        agg1 = acc1_ref[...].sum(axis=0, keepdims=True) * (1.0 / N_NBR)
        agg2 = acc2_ref[...].sum(axis=0, keepdims=True) * (1.0 / N_NBR)
        nbr1_ref[...] = jnp.dot(agg1, wn1_ref[...],
                                preferred_element_type=jnp.float32) + bn1_ref[...]
        nbr2_ref[...] = jnp.dot(agg2, wn2_ref[...],
                                preferred_element_type=jnp.float32) + bn2_ref[...]

    @pl.when(jnp.logical_and(i >= _N_RSTEPS, i < _N_RSTEPS + _N_DSTEPS))
    def _dense():
        j = i - _N_RSTEPS
        sl = pl.ds(j * _ROW_TILE, _ROW_TILE)
        out1 = a_ref[sl, :] + nbr1_ref[...]
        mu1 = jnp.mean(out1, axis=-1, keepdims=True)
        d1 = out1 - mu1
        var1 = jnp.mean(d1 * d1, axis=-1, keepdims=True)
        h1 = jnp.maximum(
            d1 * lax.rsqrt(var1 + 1e-5) * g1_ref[...] + be1_ref[...], 0.0)
        out2 = (jnp.dot(h1, ws2_ref[...], preferred_element_type=jnp.float32)
                + bs2_ref[...] + nbr2_ref[...])
        mu2 = jnp.mean(out2, axis=-1, keepdims=True)
        d2 = out2 - mu2
        var2 = jnp.mean(d2 * d2, axis=-1, keepdims=True)
        h2 = jnp.maximum(
            d2 * lax.rsqrt(var2 + 1e-5) * g2_ref[...] + be2_ref[...], 0.0)
        a_ref[sl, :] = h2
        ssq_ref[...] += jnp.sum(h2 * h2, axis=0, keepdims=True)

    @pl.when(i >= _N_RSTEPS + _N_DSTEPS)
    def _normalize():
        j = i - _N_RSTEPS - _N_DSTEPS
        scale = 1.0 / jnp.maximum(jnp.sqrt(ssq_ref[...]), 1e-12)
        out_ref[...] = a_ref[pl.ds(j * _ROW_TILE, _ROW_TILE), :] * scale


def kernel(node_feat, neighbor_feats_l1, neighbor_feats_l2, W_self1, b_self1,
           W_nbr1, b_nbr1, g1, be1, W_self2, b_self2, W_nbr2, b_nbr2, g2, be2):
    f32 = jnp.float32
    row = lambda v: v.reshape(1, -1)
    n_steps = _N_RSTEPS + 2 * _N_DSTEPS
    last_r = _N_RSTEPS - 1
    last_d = _N_DSTEPS - 1
    norm0 = _N_RSTEPS + _N_DSTEPS

    h2 = pl.pallas_call(
        _body,
        grid=(n_steps,),
        in_specs=[
            pl.BlockSpec((_R_CHUNK, FEATURE_DIM),
                         lambda i: (jnp.minimum(i, last_r), 0)),
            pl.BlockSpec((_R_CHUNK, HIDDEN_DIM),
                         lambda i: (jnp.minimum(i, last_r), 0)),
            pl.BlockSpec((_ROW_TILE, FEATURE_DIM),
                         lambda i: (jnp.minimum(i, last_d), 0)),
            pl.BlockSpec((FEATURE_DIM, HIDDEN_DIM), lambda i: (0, 0)),
            pl.BlockSpec((1, HIDDEN_DIM), lambda i: (0, 0)),
            pl.BlockSpec((HIDDEN_DIM, EMBED_DIM), lambda i: (0, 0)),
            pl.BlockSpec((1, EMBED_DIM), lambda i: (0, 0)),
            pl.BlockSpec((FEATURE_DIM, HIDDEN_DIM), lambda i: (0, 0)),
            pl.BlockSpec((1, HIDDEN_DIM), lambda i: (0, 0)),
            pl.BlockSpec((1, HIDDEN_DIM), lambda i: (0, 0)),
            pl.BlockSpec((1, HIDDEN_DIM), lambda i: (0, 0)),
            pl.BlockSpec((HIDDEN_DIM, EMBED_DIM), lambda i: (0, 0)),
            pl.BlockSpec((1, EMBED_DIM), lambda i: (0, 0)),
            pl.BlockSpec((1, EMBED_DIM), lambda i: (0, 0)),
            pl.BlockSpec((1, EMBED_DIM), lambda i: (0, 0)),
        ],
        out_specs=pl.BlockSpec(
            (_ROW_TILE, EMBED_DIM),
            lambda i: (jnp.clip(i - norm0, 0, last_d), 0)),
        out_shape=jax.ShapeDtypeStruct((N_NODES, EMBED_DIM), f32),
        scratch_shapes=[
            pltpu.VMEM((8, FEATURE_DIM), f32),
            pltpu.VMEM((8, HIDDEN_DIM), f32),
            pltpu.VMEM((N_NODES, HIDDEN_DIM), f32),
            pltpu.VMEM((1, EMBED_DIM), f32),
            pltpu.VMEM((1, HIDDEN_DIM), f32),
            pltpu.VMEM((1, EMBED_DIM), f32),
        ],
    )(neighbor_feats_l1, neighbor_feats_l2, node_feat,
      W_nbr1, row(b_nbr1), W_nbr2, row(b_nbr2),
      W_self1, row(b_self1), row(g1), row(be1),
      W_self2, row(b_self2), row(g2), row(be2))

    return h2


# speedup vs baseline: 1.0625x; 1.0625x over previous
import jax
import jax.numpy as jnp
from jax import lax
from jax.experimental import pallas as pl
from jax.experimental.pallas import tpu as pltpu

FEATURE_DIM = 128
HIDDEN_DIM = 256
EMBED_DIM = 256
N_NODES = 10000
N_NBR = 160000

_R_CHUNK = 4000
_N_RSTEPS = N_NBR // (2 * _R_CHUNK)
_ROW_TILE = 2000
_N_DSTEPS = N_NODES // _ROW_TILE


def _body(l1_ref, l1b_ref, l2_ref, l2b_ref, nf_ref,
          wn1_ref, bn1_ref, wn2_ref, bn2_ref,
          ws1_ref, bs1_ref, g1_ref, be1_ref,
          ws2_ref, bs2_ref, g2_ref, be2_ref,
          out_ref, acc1_ref, acc2_ref, a_ref, ssq_ref, nbr1_ref, nbr2_ref):
    i = pl.program_id(0)

    @pl.when(i == 0)
    def _init():
        acc1_ref[...] = jnp.zeros_like(acc1_ref)
        acc2_ref[...] = jnp.zeros_like(acc2_ref)
        ssq_ref[...] = jnp.zeros_like(ssq_ref)

    @pl.when(i < _N_RSTEPS)
    def _reduce():
        acc1_ref[...] += (
            l1_ref[...].reshape(_R_CHUNK // 8, 8, FEATURE_DIM).sum(axis=0)
            + l1b_ref[...].reshape(_R_CHUNK // 8, 8, FEATURE_DIM).sum(axis=0))
        acc2_ref[...] += (
            l2_ref[...].reshape(_R_CHUNK // 8, 8, HIDDEN_DIM).sum(axis=0)
            + l2b_ref[...].reshape(_R_CHUNK // 8, 8, HIDDEN_DIM).sum(axis=0))

    @pl.when(i < _N_DSTEPS)
    def _precompute_a():
        a_ref[pl.ds(i * _ROW_TILE, _ROW_TILE), :] = (
            jnp.dot(nf_ref[...], ws1_ref[...],
                    preferred_element_type=jnp.float32) + bs1_ref[...])

    @pl.when(i == _N_RSTEPS - 1)
    def _finalize_aggs():
        agg1 = acc1_ref[...].sum(axis=0, keepdims=True) * (1.0 / N_NBR)
        agg2 = acc2_ref[...].sum(axis=0, keepdims=True) * (1.0 / N_NBR)
        nbr1_ref[...] = jnp.dot(agg1, wn1_ref[...],
                                preferred_element_type=jnp.float32) + bn1_ref[...]
        nbr2_ref[...] = jnp.dot(agg2, wn2_ref[...],
                                preferred_element_type=jnp.float32) + bn2_ref[...]

    @pl.when(jnp.logical_and(i >= _N_RSTEPS, i < _N_RSTEPS + _N_DSTEPS))
    def _dense():
        j = i - _N_RSTEPS
        sl = pl.ds(j * _ROW_TILE, _ROW_TILE)
        out1 = a_ref[sl, :] + nbr1_ref[...]
        mu1 = jnp.mean(out1, axis=-1, keepdims=True)
        d1 = out1 - mu1
        var1 = jnp.mean(d1 * d1, axis=-1, keepdims=True)
        h1 = jnp.maximum(
            d1 * lax.rsqrt(var1 + 1e-5) * g1_ref[...] + be1_ref[...], 0.0)
        out2 = (jnp.dot(h1, ws2_ref[...], preferred_element_type=jnp.float32)
                + bs2_ref[...] + nbr2_ref[...])
        mu2 = jnp.mean(out2, axis=-1, keepdims=True)
        d2 = out2 - mu2
        var2 = jnp.mean(d2 * d2, axis=-1, keepdims=True)
        h2 = jnp.maximum(
            d2 * lax.rsqrt(var2 + 1e-5) * g2_ref[...] + be2_ref[...], 0.0)
        a_ref[sl, :] = h2
        ssq_ref[...] += jnp.sum(h2 * h2, axis=0, keepdims=True)

    @pl.when(i >= _N_RSTEPS + _N_DSTEPS)
    def _normalize():
        j = i - _N_RSTEPS - _N_DSTEPS
        scale = 1.0 / jnp.maximum(jnp.sqrt(ssq_ref[...]), 1e-12)
        out_ref[...] = a_ref[pl.ds(j * _ROW_TILE, _ROW_TILE), :] * scale


def kernel(node_feat, neighbor_feats_l1, neighbor_feats_l2, W_self1, b_self1,
           W_nbr1, b_nbr1, g1, be1, W_self2, b_self2, W_nbr2, b_nbr2, g2, be2):
    f32 = jnp.float32
    row = lambda v: v.reshape(1, -1)
    n_steps = _N_RSTEPS + 2 * _N_DSTEPS
    last_r = _N_RSTEPS - 1
    last_d = _N_DSTEPS - 1
    norm0 = _N_RSTEPS + _N_DSTEPS

    h2 = pl.pallas_call(
        _body,
        grid=(n_steps,),
        in_specs=[
            pl.BlockSpec((_R_CHUNK, FEATURE_DIM),
                         lambda i: (jnp.minimum(i, last_r), 0)),
            pl.BlockSpec((_R_CHUNK, FEATURE_DIM),
                         lambda i: (jnp.minimum(i, last_r) + _N_RSTEPS, 0)),
            pl.BlockSpec((_R_CHUNK, HIDDEN_DIM),
                         lambda i: (jnp.minimum(i, last_r), 0)),
            pl.BlockSpec((_R_CHUNK, HIDDEN_DIM),
                         lambda i: (jnp.minimum(i, last_r) + _N_RSTEPS, 0)),
            pl.BlockSpec((_ROW_TILE, FEATURE_DIM),
                         lambda i: (jnp.minimum(i, last_d), 0)),
            pl.BlockSpec((FEATURE_DIM, HIDDEN_DIM), lambda i: (0, 0)),
            pl.BlockSpec((1, HIDDEN_DIM), lambda i: (0, 0)),
            pl.BlockSpec((HIDDEN_DIM, EMBED_DIM), lambda i: (0, 0)),
            pl.BlockSpec((1, EMBED_DIM), lambda i: (0, 0)),
            pl.BlockSpec((FEATURE_DIM, HIDDEN_DIM), lambda i: (0, 0)),
            pl.BlockSpec((1, HIDDEN_DIM), lambda i: (0, 0)),
            pl.BlockSpec((1, HIDDEN_DIM), lambda i: (0, 0)),
            pl.BlockSpec((1, HIDDEN_DIM), lambda i: (0, 0)),
            pl.BlockSpec((HIDDEN_DIM, EMBED_DIM), lambda i: (0, 0)),
            pl.BlockSpec((1, EMBED_DIM), lambda i: (0, 0)),
            pl.BlockSpec((1, EMBED_DIM), lambda i: (0, 0)),
            pl.BlockSpec((1, EMBED_DIM), lambda i: (0, 0)),
        ],
        out_specs=pl.BlockSpec(
            (_ROW_TILE, EMBED_DIM),
            lambda i: (jnp.clip(i - norm0, 0, last_d), 0)),
        out_shape=jax.ShapeDtypeStruct((N_NODES, EMBED_DIM), f32),
        scratch_shapes=[
            pltpu.VMEM((8, FEATURE_DIM), f32),
            pltpu.VMEM((8, HIDDEN_DIM), f32),
            pltpu.VMEM((N_NODES, HIDDEN_DIM), f32),
            pltpu.VMEM((1, EMBED_DIM), f32),
            pltpu.VMEM((1, HIDDEN_DIM), f32),
            pltpu.VMEM((1, EMBED_DIM), f32),
        ],
    )(neighbor_feats_l1, neighbor_feats_l1, neighbor_feats_l2,
      neighbor_feats_l2, node_feat,
      W_nbr1, row(b_nbr1), W_nbr2, row(b_nbr2),
      W_self1, row(b_self1), row(g1), row(be1),
      W_self2, row(b_self2), row(g2), row(be2))

    return h2
